# baseline (speedup 1.0000x reference)
import functools

import jax
import jax.numpy as jnp
from jax import lax
from jax.experimental import pallas as pl
from jax.experimental.pallas import tpu as pltpu
from jax.experimental.pallas import tpu_sc as plsc

_N_BINS = 10
_BB = 32
_NC = 2
_NS = 16
_NW = _NC * _NS
_L = 16
_SC_B = 128
_SPW = _SC_B // _NW


def _tc_hist_kernel(pred_ref, truth_ref, out_ref):
    x = jnp.swapaxes(pred_ref[...], 1, 2)
    tv = truth_ref[...]
    t = tv[:, None, :]

    mask = x < t
    r = jnp.sum(mask.astype(jnp.float32), axis=1).astype(jnp.int32)
    a = jnp.max(jnp.where(mask, x, -jnp.inf), axis=1)
    b = jnp.min(jnp.where(mask, jnp.inf, x), axis=1)

    base = jnp.where(r >= 1, 1 + jnp.minimum((r - 1) // 10, 9), 0)
    base = base + jnp.where(r == 100, 1, 0)

    amb = (r % 10 == 0) & (r >= 10) & (r <= 90)
    kf = (r // 10).astype(jnp.float32)
    qv = kf * jnp.float32(0.1)
    idx = qv * jnp.float32(99.0)
    hw = idx - jnp.floor(idx)
    lw = jnp.float32(1.0) - hw
    interp = a * lw + b * hw
    m = base + jnp.where(amb & (interp < tv), 1, 0)

    bin0 = jnp.clip(m, 1, _N_BINS) - 1

    one_hot = (bin0[:, :, None] == jax.lax.broadcasted_iota(
        jnp.int32, (1, 1, _N_BINS), 2)).astype(jnp.float32)
    hist = jnp.sum(one_hot, axis=(0, 1))

    @pl.when(pl.program_id(0) == 0)
    def _init():
        out_ref[...] = jnp.zeros_like(out_ref)

    out_ref[0, :] += hist


def _tc_hist(prediction, truth, nb_tc):
    return pl.pallas_call(
        _tc_hist_kernel,
        grid=(nb_tc // _BB,),
        in_specs=[
            pl.BlockSpec((_BB, 256, 100), lambda i: (i, 0, 0)),
            pl.BlockSpec((_BB, 256), lambda i: (i, 0)),
        ],
        out_specs=pl.BlockSpec((1, _N_BINS), lambda i: (0, 0)),
        out_shape=jax.ShapeDtypeStruct((1, _N_BINS), jnp.float32),
    )(prediction, truth)


_sc_mesh = plsc.VectorSubcoreMesh(
    core_axis_name="c", subcore_axis_name="s", num_cores=_NC, num_subcores=_NS)


def _tc_token_kernel(t_ref, o_ref):
    o_ref[...] = t_ref[0:1, 0:_L] * jnp.float32(0.0)


def _tc_token(truth):
    return pl.pallas_call(
        _tc_token_kernel,
        out_shape=jax.ShapeDtypeStruct((1, _L), jnp.float32),
    )(truth)


@functools.partial(
    pl.kernel,
    out_type=jax.ShapeDtypeStruct((_NW, _N_BINS, _L), jnp.float32),
    mesh=_sc_mesh,
    scratch_types=[
        pltpu.VMEM((256, 100), jnp.float32),
        pltpu.VMEM((256,), jnp.float32),
        pltpu.VMEM((_N_BINS, _L), jnp.float32),
        pltpu.VMEM((1, _L), jnp.float32),
    ],
    compiler_params=pltpu.CompilerParams(needs_layout_passes=False),
)
def _sc_qice(pred_hbm, truth_hbm, tok_hbm, out_hbm, xbuf, tbuf, hist_v, tok_v):
    nb = pred_hbm.shape[0]
    wid = lax.axis_index("s") * _NC + lax.axis_index("c")
    b0 = nb - _SC_B + wid * _SPW

    lanes = lax.iota(jnp.int32, _L)
    fone = jnp.ones((_L,), jnp.float32)
    fzero = jnp.zeros((_L,), jnp.float32)
    ione = jnp.ones((_L,), jnp.int32)
    izero = jnp.zeros((_L,), jnp.int32)
    ninf = jnp.full((_L,), -jnp.inf, jnp.float32)
    pinf = jnp.full((_L,), jnp.inf, jnp.float32)

    def batch_body(bi, accs):
        b = b0 + bi
        pltpu.sync_copy(pred_hbm.at[b], xbuf)
        pltpu.sync_copy(truth_hbm.at[b], tbuf)

        def group_body(g, accs):
            d_vec = g * _L + lanes
            t16 = tbuf[pl.ds(g * _L, _L)]

            r, a, bv = izero, ninf, pinf
            j_vec = izero
            for _ in range(100):
                x16 = plsc.load_gather(xbuf, [d_vec, j_vec])
                mlt = x16 < t16
                r = r + jnp.where(mlt, ione, izero)
                a = jnp.maximum(a, jnp.where(mlt, x16, ninf))
                bv = jnp.minimum(bv, jnp.where(mlt, pinf, x16))
                j_vec = j_vec + ione

            base = jnp.where(r >= 1, 1 + jnp.minimum((r - 1) // 10, 9), 0)
            base = base + jnp.where(r == 100, 1, 0)
            amb = (r % 10 == 0) & (r >= 10) & (r <= 90)
            kf = (r // 10).astype(jnp.float32)
            qv = kf * jnp.float32(0.1)
            idxq = qv * jnp.float32(99.0)
            hw = idxq - (r - 1).astype(jnp.float32)
            lw = jnp.float32(1.0) - hw
            interp = a * lw + bv * hw
            m = base + jnp.where(amb & (interp < t16), 1, 0)
            bin0 = jnp.clip(m, 1, _N_BINS) - 1

            return tuple(
                acc + jnp.where(bin0 == k, fone, fzero)
                for k, acc in enumerate(accs))

        return lax.fori_loop(0, 256 // _L, group_body, accs)

    accs = lax.fori_loop(
        0, _SPW, batch_body,
        tuple(jnp.zeros((_L,), jnp.float32) for _ in range(_N_BINS)))

    pltpu.sync_copy(tok_hbm, tok_v)
    tok = tok_v[0, :]
    for k, acc in enumerate(accs):
        hist_v[k, :] = acc + tok
    pltpu.sync_copy(hist_v, out_hbm.at[wid])


@jax.jit
def kernel(prediction, truth):
    nb = prediction.shape[0]
    tok = _tc_token(truth)
    sc_part = _sc_qice(prediction, truth, tok)
    tc_hist = _tc_hist(prediction, truth, nb - _SC_B)
    return tc_hist[0] + jnp.sum(sc_part, axis=(0, 2))


# speedup vs baseline: 1.1608x; 1.1608x over previous
import functools

import jax
import jax.numpy as jnp
from jax import lax
from jax.experimental import pallas as pl
from jax.experimental.pallas import tpu as pltpu
from jax.experimental.pallas import tpu_sc as plsc

_N_BINS = 10
_BB = 32
_NC = 2
_NS = 16
_NW = _NC * _NS
_L = 16


def _tc_bins_kernel(pred_ref, truth_ref, bins_ref):
    x = jnp.swapaxes(pred_ref[...], 1, 2)
    tv = truth_ref[...]
    t = tv[:, None, :]

    mask = x < t
    r = jnp.sum(mask.astype(jnp.float32), axis=1).astype(jnp.int32)
    a = jnp.max(jnp.where(mask, x, -jnp.inf), axis=1)
    b = jnp.min(jnp.where(mask, jnp.inf, x), axis=1)

    base = jnp.where(r >= 1, 1 + jnp.minimum((r - 1) // 10, 9), 0)
    base = base + jnp.where(r == 100, 1, 0)

    amb = (r % 10 == 0) & (r >= 10) & (r <= 90)
    kf = (r // 10).astype(jnp.float32)
    qv = kf * jnp.float32(0.1)
    idx = qv * jnp.float32(99.0)
    hw = idx - jnp.floor(idx)
    lw = jnp.float32(1.0) - hw
    interp = a * lw + b * hw
    m = base + jnp.where(amb & (interp < tv), 1, 0)

    bins_ref[...] = jnp.clip(m, 1, _N_BINS) - 1


def _tc_bins(prediction, truth):
    nb = prediction.shape[0]
    return pl.pallas_call(
        _tc_bins_kernel,
        grid=(nb // _BB,),
        in_specs=[
            pl.BlockSpec((_BB, 256, 100), lambda i: (i, 0, 0)),
            pl.BlockSpec((_BB, 256), lambda i: (i, 0)),
        ],
        out_specs=pl.BlockSpec((_BB, 256), lambda i: (i, 0)),
        out_shape=jax.ShapeDtypeStruct((nb, 256), jnp.int32),
    )(prediction, truth)


_sc_mesh = plsc.VectorSubcoreMesh(
    core_axis_name="c", subcore_axis_name="s", num_cores=_NC, num_subcores=_NS)


@functools.partial(
    pl.kernel,
    out_type=jax.ShapeDtypeStruct((_NW, _N_BINS, _L), jnp.float32),
    mesh=_sc_mesh,
    scratch_types=[
        pltpu.VMEM((16, 256), jnp.int32),
        pltpu.VMEM((_N_BINS, _L), jnp.float32),
    ],
)
def _sc_hist(bins_hbm, out_hbm, in_v, hist_v):
    wid = lax.axis_index("s") * _NC + lax.axis_index("c")
    pltpu.sync_copy(bins_hbm.at[pl.ds(wid * 16, 16)], in_v)

    one = jnp.ones((_L,), jnp.float32)
    zero = jnp.zeros((_L,), jnp.float32)

    def row_body(rr, accs):
        for cc in range(256 // _L):
            v = in_v[rr, pl.ds(cc * _L, _L)]
            accs = tuple(
                acc + jnp.where(v == k, one, zero)
                for k, acc in enumerate(accs))
        return accs

    accs = lax.fori_loop(
        0, 16, row_body,
        tuple(jnp.zeros((_L,), jnp.float32) for _ in range(_N_BINS)))

    for k, acc in enumerate(accs):
        hist_v[k, :] = acc
    pltpu.sync_copy(hist_v, out_hbm.at[wid])


@jax.jit
def kernel(prediction, truth):
    bins = _tc_bins(prediction, truth)
    part = _sc_hist(bins)
    return jnp.sum(part, axis=(0, 2))


# speedup vs baseline: 1.2320x; 1.0613x over previous
import functools

import jax
import jax.numpy as jnp
from jax import lax
from jax.experimental import pallas as pl
from jax.experimental.pallas import tpu as pltpu
from jax.experimental.pallas import tpu_sc as plsc

_N_BINS = 10
_BB = 32
_NC = 2
_NS = 16
_NW = _NC * _NS
_L = 16


def _tc_bins_kernel(pred_ref, truth_ref, bins_ref):
    x = jnp.swapaxes(pred_ref[...], 1, 2)
    tv = truth_ref[...]
    t = tv[:, None, :]

    mask = x < t
    r = jnp.sum(mask.astype(jnp.float32), axis=1).astype(jnp.int32)
    a = jnp.max(jnp.where(mask, x, -jnp.inf), axis=1)
    b = jnp.min(jnp.where(mask, jnp.inf, x), axis=1)

    base = jnp.where(r >= 1, 1 + jnp.minimum((r - 1) // 10, 9), 0)
    base = base + jnp.where(r == 100, 1, 0)

    amb = (r % 10 == 0) & (r >= 10) & (r <= 90)
    kf = (r // 10).astype(jnp.float32)
    qv = kf * jnp.float32(0.1)
    idx = qv * jnp.float32(99.0)
    hw = idx - jnp.floor(idx)
    lw = jnp.float32(1.0) - hw
    interp = a * lw + b * hw
    m = base + jnp.where(amb & (interp < tv), 1, 0)

    bins_ref[...] = jnp.clip(m, 1, _N_BINS) - 1


def _tc_bins(prediction, truth):
    nb = prediction.shape[0]
    return pl.pallas_call(
        _tc_bins_kernel,
        grid=(nb // _BB,),
        in_specs=[
            pl.BlockSpec((_BB, 256, 100), lambda i: (i, 0, 0)),
            pl.BlockSpec((_BB, 256), lambda i: (i, 0)),
        ],
        out_specs=pl.BlockSpec((_BB, 256), lambda i: (i, 0)),
        out_shape=jax.ShapeDtypeStruct((nb, 256), jnp.int32),
    )(prediction, truth)


_sc_mesh = plsc.VectorSubcoreMesh(
    core_axis_name="c", subcore_axis_name="s", num_cores=_NC, num_subcores=_NS)


@functools.partial(
    pl.kernel,
    out_type=jax.ShapeDtypeStruct((_NW, _N_BINS, _L), jnp.float32),
    mesh=_sc_mesh,
    scratch_types=[
        pltpu.VMEM((16, 256), jnp.int32),
        pltpu.VMEM((_N_BINS, _L), jnp.float32),
    ],
)
def _sc_hist(bins_hbm, out_hbm, in_v, hist_v):
    wid = lax.axis_index("s") * _NC + lax.axis_index("c")
    pltpu.sync_copy(bins_hbm.at[pl.ds(wid * 16, 16)], in_v)

    one = jnp.ones((_L,), jnp.float32)
    zero = jnp.zeros((_L,), jnp.float32)

    def row_body(rr, accs):
        def col_body(cc, accs):
            v = in_v[rr, pl.ds(cc * _L, _L)]
            return tuple(
                acc + jnp.where(v == k, one, zero)
                for k, acc in enumerate(accs))
        return lax.fori_loop(0, 256 // _L, col_body, accs)

    accs = lax.fori_loop(
        0, 16, row_body,
        tuple(jnp.zeros((_L,), jnp.float32) for _ in range(_N_BINS)))

    for k, acc in enumerate(accs):
        hist_v[k, :] = acc
    pltpu.sync_copy(hist_v, out_hbm.at[wid])


@jax.jit
def kernel(prediction, truth):
    bins = _tc_bins(prediction, truth)
    part = _sc_hist(bins)
    return jnp.sum(part, axis=(0, 2))


# speedup vs baseline: 1.2358x; 1.0030x over previous
import functools

import jax
import jax.numpy as jnp
from jax import lax
from jax.experimental import pallas as pl
from jax.experimental.pallas import tpu as pltpu
from jax.experimental.pallas import tpu_sc as plsc

_N_BINS = 10
_BB = 32
_NC = 2
_NS = 16
_NW = _NC * _NS
_L = 16


def _tc_bins_kernel(pred_ref, truth_ref, bins_ref):
    x = jnp.swapaxes(pred_ref[...], 1, 2)
    tv = truth_ref[...]
    t = tv[:, None, :]

    mask = x < t
    r = jnp.sum(mask.astype(jnp.float32), axis=1).astype(jnp.int32)
    a = jnp.max(jnp.where(mask, x, -jnp.inf), axis=1)
    b = jnp.min(jnp.where(mask, jnp.inf, x), axis=1)

    base = jnp.where(r >= 1, 1 + jnp.minimum((r - 1) // 10, 9), 0)
    base = base + jnp.where(r == 100, 1, 0)

    amb = (r % 10 == 0) & (r >= 10) & (r <= 90)
    kf = (r // 10).astype(jnp.float32)
    qv = kf * jnp.float32(0.1)
    idx = qv * jnp.float32(99.0)
    hw = idx - jnp.floor(idx)
    lw = jnp.float32(1.0) - hw
    interp = a * lw + b * hw
    m = base + jnp.where(amb & (interp < tv), 1, 0)

    bins_ref[...] = jnp.clip(m, 1, _N_BINS) - 1


def _tc_bins(prediction, truth):
    nb = prediction.shape[0]
    return pl.pallas_call(
        _tc_bins_kernel,
        grid=(nb // _BB,),
        in_specs=[
            pl.BlockSpec((_BB, 256, 100), lambda i: (i, 0, 0)),
            pl.BlockSpec((_BB, 256), lambda i: (i, 0)),
        ],
        out_specs=pl.BlockSpec((_BB, 256), lambda i: (i, 0)),
        out_shape=jax.ShapeDtypeStruct((nb, 256), jnp.int32),
    )(prediction, truth)


_sc_mesh = plsc.VectorSubcoreMesh(
    core_axis_name="c", subcore_axis_name="s", num_cores=_NC, num_subcores=_NS)


@functools.partial(
    pl.kernel,
    out_type=jax.ShapeDtypeStruct((_NW, _L), jnp.float32),
    mesh=_sc_mesh,
    scratch_types=[
        pltpu.VMEM((16, 256), jnp.int32),
        pltpu.VMEM((_L,), jnp.float32),
    ],
    compiler_params=pltpu.CompilerParams(needs_layout_passes=False),
)
def _sc_hist(bins_hbm, out_hbm, in_v, hist_v):
    wid = lax.axis_index("s") * _NC + lax.axis_index("c")
    pltpu.sync_copy(bins_hbm.at[pl.ds(wid * 16, 16)], in_v)

    hist_v[...] = jnp.zeros((_L,), jnp.float32)
    ones = jnp.ones((_L,), jnp.float32)

    def row_body(rr, carry):
        def col_body(cc, carry):
            v = in_v[rr, pl.ds(cc * _L, _L)]
            plsc.addupdate_scatter(hist_v, [v], ones)
            return carry
        return lax.fori_loop(0, 256 // _L, col_body, carry)

    lax.fori_loop(0, 16, row_body, 0)
    pltpu.sync_copy(hist_v, out_hbm.at[wid])


@jax.jit
def kernel(prediction, truth):
    bins = _tc_bins(prediction, truth)
    part = _sc_hist(bins)
    return jnp.sum(part, axis=0)[:_N_BINS]
